# Initial kernel scaffold; baseline (speedup 1.0000x reference)
import jax
import jax.numpy as jnp
from jax.experimental import pallas as pl


def kernel(x, edge_index, W_self, W_neigh, b):
    raise NotImplementedError("write your pallas kernel here")


# speedup vs baseline: 2.7147x; 2.7147x over previous
import functools

import jax
import jax.numpy as jnp
from jax import lax
from jax.experimental import pallas as pl
from jax.experimental.pallas import tpu as pltpu
from jax.experimental.pallas import tpu_sc as plsc

N = 10000
E = 160000
D = 256
DH = 128
NS = 16
CHUNK = 128
E_PAD = 163840
CHUNKS_PER_SUB = E_PAD // (NS * CHUNK)
EDGES_PER_SUB = E_PAD // NS
AGG_ROWS = 10240
ROWS_PER_SUB = AGG_ROWS // NS
WB = 128

_sc_mesh = plsc.VectorSubcoreMesh(core_axis_name="c", subcore_axis_name="s")


@functools.partial(
    pl.kernel,
    mesh=_sc_mesh,
    out_type=jax.ShapeDtypeStruct((2 * AGG_ROWS, DH), jnp.float32),
    scratch_types=[
        pltpu.VMEM((CHUNK,), jnp.int32),
        pltpu.VMEM((CHUNK,), jnp.int32),
        pltpu.VMEM((CHUNK, DH), jnp.float32),
        pltpu.VMEM_SHARED((AGG_ROWS, DH), jnp.float32),
    ],
)
def _sc_aggregate(xr, srcb, dst1, zeros_in, agg01,
                  sidx, didx, rows, agg_sh):
    c = lax.axis_index("c")
    s = lax.axis_index("s")

    pltpu.sync_copy(zeros_in, rows)
    base = s * ROWS_PER_SUB
    for k in range(ROWS_PER_SUB // WB):
        pltpu.sync_copy(rows, agg_sh.at[pl.ds(base + k * WB, WB)])

    plsc.subcore_barrier()

    def _chunk(g, carry):
        ebase = s * EDGES_PER_SUB + g * CHUNK
        pltpu.sync_copy(srcb.at[pl.ds(c * E_PAD + ebase, CHUNK)], sidx)
        pltpu.sync_copy(dst1.at[pl.ds(ebase, CHUNK)], didx)
        pltpu.sync_copy(xr.at[sidx], rows)
        pltpu.sync_copy(rows, agg_sh.at[didx], add=True)
        return carry
    lax.fori_loop(0, CHUNKS_PER_SUB, _chunk, 0)

    plsc.subcore_barrier()

    for k in range(ROWS_PER_SUB // WB):
        r0 = base + k * WB
        pltpu.sync_copy(agg_sh.at[pl.ds(r0, WB)], rows)
        pltpu.sync_copy(rows, agg01.at[pl.ds(c * AGG_ROWS + r0, WB)])


DEG_W = 128
DEG_CHUNKS = E_PAD // (2 * NS * CHUNK)


@functools.partial(
    pl.kernel,
    mesh=_sc_mesh,
    out_type=jax.ShapeDtypeStruct((2 * AGG_ROWS, DEG_W), jnp.float32),
    scratch_types=[
        pltpu.VMEM((CHUNK,), jnp.int32),
        pltpu.VMEM((WB, DEG_W), jnp.float32),
        pltpu.VMEM_SHARED((AGG_ROWS, DEG_W), jnp.float32),
    ],
)
def _sc_degree(dst1, zeros16_in, ones_in, deg01, didx, ones_v, deg_sh):
    c = lax.axis_index("c")
    s = lax.axis_index("s")

    base = s * ROWS_PER_SUB
    pltpu.sync_copy(zeros16_in, ones_v)
    for k in range(ROWS_PER_SUB // WB):
        pltpu.sync_copy(ones_v, deg_sh.at[pl.ds(base + k * WB, WB)])
    pltpu.sync_copy(ones_in, ones_v)
    plsc.subcore_barrier()

    def _chunk(g, carry):
        ebase = c * (E_PAD // 2) + s * (E_PAD // (2 * NS)) + g * CHUNK
        pltpu.sync_copy(dst1.at[pl.ds(ebase, CHUNK)], didx)
        pltpu.sync_copy(ones_v, deg_sh.at[didx], add=True)
        return carry
    lax.fori_loop(0, DEG_CHUNKS, _chunk, 0)

    plsc.subcore_barrier()

    for k in range(ROWS_PER_SUB // WB):
        r0 = base + k * WB
        pltpu.sync_copy(deg_sh.at[pl.ds(r0, WB)], ones_v)
        pltpu.sync_copy(ones_v, deg01.at[pl.ds(c * AGG_ROWS + r0, WB)])


_BLK = 1000


def _tc_body(x_ref, ws_ref, wn0_ref, wn1_ref, b_ref, a0_ref, a1_ref,
             d0_ref, d1_ref, out_ref):
    inv = 1.0 / jnp.maximum(d0_ref[...] + d1_ref[...], 1.0)
    dn = (((1,), (1,)), ((), ()))
    acc = lax.dot_general(x_ref[...], ws_ref[...], dn,
                          preferred_element_type=jnp.float32)
    acc += lax.dot_general(a0_ref[...] * inv, wn0_ref[...], dn,
                           preferred_element_type=jnp.float32)
    acc += lax.dot_general(a1_ref[...] * inv, wn1_ref[...], dn,
                           preferred_element_type=jnp.float32)
    out_ref[...] = acc + b_ref[...][None, :]


def _tc_combine(x, W_self, Wn0, Wn1, b, agg0, agg1, d0, d1):
    grid = (N // _BLK,)
    return pl.pallas_call(
        _tc_body,
        grid=grid,
        in_specs=[
            pl.BlockSpec((_BLK, D), lambda i: (i, 0)),
            pl.BlockSpec((D, D), lambda i: (0, 0)),
            pl.BlockSpec((D, DH), lambda i: (0, 0)),
            pl.BlockSpec((D, DH), lambda i: (0, 0)),
            pl.BlockSpec((D,), lambda i: (0,)),
            pl.BlockSpec((_BLK, DH), lambda i: (i, 0)),
            pl.BlockSpec((_BLK, DH), lambda i: (i, 0)),
            pl.BlockSpec((_BLK, 1), lambda i: (i, 0)),
            pl.BlockSpec((_BLK, 1), lambda i: (i, 0)),
        ],
        out_specs=pl.BlockSpec((_BLK, D), lambda i: (i, 0)),
        out_shape=jax.ShapeDtypeStruct((N, D), jnp.float32),
    )(x, W_self, Wn0, Wn1, b, agg0, agg1, d0, d1)


def kernel(x, edge_index, W_self, W_neigh, b):
    src = edge_index[0]
    dst = edge_index[1]
    pad = E_PAD - E
    src1 = jnp.concatenate([src, jnp.zeros((pad,), jnp.int32)])
    dst1 = jnp.concatenate([dst, jnp.full((pad,), N, jnp.int32)])
    xr = x.reshape(2 * N, DH)
    srcb = jnp.concatenate([2 * src1, 2 * src1 + 1])
    zeros_in = jnp.zeros((CHUNK, DH), jnp.float32)
    ones_in = jnp.ones((WB, DEG_W), jnp.float32)
    zeros16_in = jnp.zeros((WB, DEG_W), jnp.float32)

    agg01 = _sc_aggregate(xr, srcb, dst1, zeros_in)
    deg01 = _sc_degree(dst1, zeros16_in, ones_in)
    agg0 = agg01[:N]
    agg1 = agg01[AGG_ROWS:AGG_ROWS + N]
    d0 = deg01[:N, :1]
    d1 = deg01[AGG_ROWS:AGG_ROWS + N, :1]

    Wn0 = W_neigh[:, :DH]
    Wn1 = W_neigh[:, DH:]
    return _tc_combine(x, W_self, Wn0, Wn1, b, agg0, agg1, d0, d1)


# speedup vs baseline: 3.2982x; 1.2149x over previous
import functools

import jax
import jax.numpy as jnp
from jax import lax
from jax.experimental import pallas as pl
from jax.experimental.pallas import tpu as pltpu
from jax.experimental.pallas import tpu_sc as plsc

N = 10000
E = 160000
D = 256
DH = 128
NS = 16
CHUNK = 128
E_PAD = 163840
CHUNKS_PER_SUB = E_PAD // (NS * CHUNK)
EDGES_PER_SUB = E_PAD // NS
AGG_ROWS = 10240
ROWS_PER_SUB = AGG_ROWS // NS
WB = 128

_sc_mesh = plsc.VectorSubcoreMesh(core_axis_name="c", subcore_axis_name="s")


@functools.partial(
    pl.kernel,
    mesh=_sc_mesh,
    out_type=jax.ShapeDtypeStruct((2 * AGG_ROWS, DH), jnp.float32),
    scratch_types=[
        pltpu.VMEM((CHUNK,), jnp.int32),
        pltpu.VMEM((CHUNK,), jnp.int32),
        pltpu.VMEM((CHUNK,), jnp.int32),
        pltpu.VMEM((CHUNK,), jnp.int32),
        pltpu.VMEM((CHUNK, DH), jnp.float32),
        pltpu.VMEM((CHUNK, DH), jnp.float32),
        pltpu.SemaphoreType.DMA,
        pltpu.SemaphoreType.DMA,
        pltpu.SemaphoreType.DMA,
        pltpu.SemaphoreType.DMA,
        pltpu.VMEM_SHARED((AGG_ROWS, DH), jnp.float32),
    ],
)
def _sc_aggregate(xr, srcb, dst1, zeros_in, agg01,
                  sidxA, didxA, sidxB, didxB, rowsA, rowsB,
                  semIA, semIB, semGA, semGB, agg_sh):
    c = lax.axis_index("c")
    s = lax.axis_index("s")

    pltpu.sync_copy(zeros_in, rowsA)
    base = s * ROWS_PER_SUB
    for k in range(ROWS_PER_SUB // WB):
        pltpu.sync_copy(rowsA, agg_sh.at[pl.ds(base + k * WB, WB)])

    plsc.subcore_barrier()

    ebase0 = s * EDGES_PER_SUB
    sbase0 = c * E_PAD + ebase0

    def _src_at(g):
        return srcb.at[pl.ds(sbase0 + g * CHUNK, CHUNK)]

    def _dst_at(g):
        return dst1.at[pl.ds(ebase0 + g * CHUNK, CHUNK)]

    pltpu.sync_copy(_src_at(0), sidxA)
    pltpu.sync_copy(_dst_at(0), didxA)
    pltpu.sync_copy(_src_at(1), sidxB)
    pltpu.sync_copy(_dst_at(1), didxB)
    pltpu.async_copy(xr.at[sidxA], rowsA, semGA)
    pltpu.async_copy(xr.at[sidxB], rowsB, semGB)

    def _body(t, carry):
        pltpu.make_async_copy(xr.at[sidxA], rowsA, semGA).wait()
        pltpu.sync_copy(rowsA, agg_sh.at[didxA], add=True)
        cp1 = pltpu.make_async_copy(_src_at(2 * t + 2), sidxA, semIA)
        cp2 = pltpu.make_async_copy(_dst_at(2 * t + 2), didxA, semIA)
        cp1.start()
        cp2.start()
        pltpu.make_async_copy(xr.at[sidxB], rowsB, semGB).wait()
        pltpu.sync_copy(rowsB, agg_sh.at[didxB], add=True)
        cp3 = pltpu.make_async_copy(_src_at(2 * t + 3), sidxB, semIB)
        cp4 = pltpu.make_async_copy(_dst_at(2 * t + 3), didxB, semIB)
        cp3.start()
        cp4.start()
        cp1.wait()
        cp2.wait()
        pltpu.async_copy(xr.at[sidxA], rowsA, semGA)
        cp3.wait()
        cp4.wait()
        pltpu.async_copy(xr.at[sidxB], rowsB, semGB)
        return carry
    lax.fori_loop(0, CHUNKS_PER_SUB // 2 - 1, _body, 0)

    pltpu.make_async_copy(xr.at[sidxA], rowsA, semGA).wait()
    pltpu.sync_copy(rowsA, agg_sh.at[didxA], add=True)
    pltpu.make_async_copy(xr.at[sidxB], rowsB, semGB).wait()
    pltpu.sync_copy(rowsB, agg_sh.at[didxB], add=True)

    plsc.subcore_barrier()

    for k in range(ROWS_PER_SUB // WB):
        r0 = base + k * WB
        pltpu.sync_copy(agg_sh.at[pl.ds(r0, WB)], rowsA)
        pltpu.sync_copy(rowsA, agg01.at[pl.ds(c * AGG_ROWS + r0, WB)])


DEG_W = 128
DEG_CHUNKS = E_PAD // (2 * NS * CHUNK)


@functools.partial(
    pl.kernel,
    mesh=_sc_mesh,
    out_type=jax.ShapeDtypeStruct((2 * AGG_ROWS, DEG_W), jnp.float32),
    scratch_types=[
        pltpu.VMEM((CHUNK,), jnp.int32),
        pltpu.VMEM((WB, DEG_W), jnp.float32),
        pltpu.VMEM_SHARED((AGG_ROWS, DEG_W), jnp.float32),
    ],
)
def _sc_degree(dst1, zeros16_in, ones_in, deg01, didx, ones_v, deg_sh):
    c = lax.axis_index("c")
    s = lax.axis_index("s")

    base = s * ROWS_PER_SUB
    pltpu.sync_copy(zeros16_in, ones_v)
    for k in range(ROWS_PER_SUB // WB):
        pltpu.sync_copy(ones_v, deg_sh.at[pl.ds(base + k * WB, WB)])
    pltpu.sync_copy(ones_in, ones_v)
    plsc.subcore_barrier()

    def _chunk(g, carry):
        ebase = c * (E_PAD // 2) + s * (E_PAD // (2 * NS)) + g * CHUNK
        pltpu.sync_copy(dst1.at[pl.ds(ebase, CHUNK)], didx)
        pltpu.sync_copy(ones_v, deg_sh.at[didx], add=True)
        return carry
    lax.fori_loop(0, DEG_CHUNKS, _chunk, 0)

    plsc.subcore_barrier()

    for k in range(ROWS_PER_SUB // WB):
        r0 = base + k * WB
        pltpu.sync_copy(deg_sh.at[pl.ds(r0, WB)], ones_v)
        pltpu.sync_copy(ones_v, deg01.at[pl.ds(c * AGG_ROWS + r0, WB)])


_BLK = 1000


def _tc_body(x_ref, ws_ref, wn0_ref, wn1_ref, b_ref, a0_ref, a1_ref,
             d0_ref, d1_ref, out_ref):
    inv = 1.0 / jnp.maximum(d0_ref[...] + d1_ref[...], 1.0)
    dn = (((1,), (1,)), ((), ()))
    acc = lax.dot_general(x_ref[...], ws_ref[...], dn,
                          preferred_element_type=jnp.float32)
    acc += lax.dot_general(a0_ref[...] * inv, wn0_ref[...], dn,
                           preferred_element_type=jnp.float32)
    acc += lax.dot_general(a1_ref[...] * inv, wn1_ref[...], dn,
                           preferred_element_type=jnp.float32)
    out_ref[...] = acc + b_ref[...][None, :]


def _tc_combine(x, W_self, Wn0, Wn1, b, agg0, agg1, d0, d1):
    grid = (N // _BLK,)
    return pl.pallas_call(
        _tc_body,
        grid=grid,
        in_specs=[
            pl.BlockSpec((_BLK, D), lambda i: (i, 0)),
            pl.BlockSpec((D, D), lambda i: (0, 0)),
            pl.BlockSpec((D, DH), lambda i: (0, 0)),
            pl.BlockSpec((D, DH), lambda i: (0, 0)),
            pl.BlockSpec((D,), lambda i: (0,)),
            pl.BlockSpec((_BLK, DH), lambda i: (i, 0)),
            pl.BlockSpec((_BLK, DH), lambda i: (i, 0)),
            pl.BlockSpec((_BLK, 1), lambda i: (i, 0)),
            pl.BlockSpec((_BLK, 1), lambda i: (i, 0)),
        ],
        out_specs=pl.BlockSpec((_BLK, D), lambda i: (i, 0)),
        out_shape=jax.ShapeDtypeStruct((N, D), jnp.float32),
    )(x, W_self, Wn0, Wn1, b, agg0, agg1, d0, d1)


def kernel(x, edge_index, W_self, W_neigh, b):
    src = edge_index[0]
    dst = edge_index[1]
    pad = E_PAD - E
    src1 = jnp.concatenate([src, jnp.zeros((pad,), jnp.int32)])
    dst1 = jnp.concatenate([dst, jnp.full((pad,), N, jnp.int32)])
    xr = x.reshape(2 * N, DH)
    srcb = jnp.concatenate([2 * src1, 2 * src1 + 1])
    zeros_in = jnp.zeros((CHUNK, DH), jnp.float32)
    ones_in = jnp.ones((WB, DEG_W), jnp.float32)
    zeros16_in = jnp.zeros((WB, DEG_W), jnp.float32)

    agg01 = _sc_aggregate(xr, srcb, dst1, zeros_in)
    deg01 = _sc_degree(dst1, zeros16_in, ones_in)
    agg0 = agg01[:N]
    agg1 = agg01[AGG_ROWS:AGG_ROWS + N]
    d0 = deg01[:N, :1]
    d1 = deg01[AGG_ROWS:AGG_ROWS + N, :1]

    Wn0 = W_neigh[:, :DH]
    Wn1 = W_neigh[:, DH:]
    return _tc_combine(x, W_self, Wn0, Wn1, b, agg0, agg1, d0, d1)
